# baseline (speedup 1.0000x reference)
import functools
import jax
import jax.numpy as jnp
from jax import lax
from jax.experimental import pallas as pl
from jax.experimental.pallas import tpu as pltpu
from jax.experimental.pallas import tpu_sc as plsc

_NUM_ACTIONS = 1000000
_LATENT_DIM = 16
_BATCH = 16384

_NC = 2
_NS = 16
_NW = _NC * _NS
_B_PER_W = _BATCH // _NW
_L = 16

_FULL_LT = _NUM_ACTIONS // 128
_LT_PER_W = _FULL_LT // _NW
_EXTRA_LT = _FULL_LT - _NW * _LT_PER_W
_TAIL_START = _FULL_LT * 128
_FLAT_WORDS = _FULL_LT * _LATENT_DIM * 128

_mesh = plsc.VectorSubcoreMesh(core_axis_name="c", subcore_axis_name="s")


@functools.partial(
    pl.kernel,
    mesh=_mesh,
    out_type=jax.ShapeDtypeStruct((_FULL_LT, _LATENT_DIM, 128), jnp.float32),
    scratch_types=[
        pltpu.VMEM((_LATENT_DIM, 4096), jnp.float32),
        pltpu.VMEM((_LATENT_DIM, 4096), jnp.float32),
        pltpu.SemaphoreType.DMA,
        pltpu.SemaphoreType.DMA,
    ],
    compiler_params=pltpu.CompilerParams(
        needs_layout_passes=False, skip_device_barrier=True
    ),
)
def _detile_kernel(table_t_hbm, flat_hbm, slab0_v, slab1_v, rsem, wsem):
    wid = lax.axis_index("s") * _NC + lax.axis_index("c")
    base_lt = wid * _LT_PER_W
    n_slabs = _LT_PER_W // 32
    rem_lt = _LT_PER_W - 32 * n_slabs
    slabs = [slab0_v, slab1_v]

    def read_slab(lt0, n_lt, buf):
        pltpu.async_copy(
            table_t_hbm.at[:, pl.ds(lt0 * 128, n_lt * 128)],
            buf.at[:, pl.ds(0, n_lt * 128)],
            rsem,
        )

    def wait_read(lt0, n_lt, buf):
        pltpu.make_async_copy(
            table_t_hbm.at[:, pl.ds(lt0 * 128, n_lt * 128)],
            buf.at[:, pl.ds(0, n_lt * 128)],
            rsem,
        ).wait()

    def write_slab(lt0, n_lt, buf):
        for j in range(n_lt):
            pltpu.async_copy(
                buf.at[:, pl.ds(j * 128, 128)], flat_hbm.at[lt0 + j], wsem
            )

    def drain_writes(lt0, n_lt, buf):
        for j in range(n_lt):
            pltpu.make_async_copy(
                buf.at[:, pl.ds(j * 128, 128)], flat_hbm.at[lt0 + j], wsem
            ).wait()

    read_slab(base_lt, 32, slabs[0])
    read_slab(base_lt + 32, 32, slabs[1])
    for k in range(n_slabs):
        wait_read(base_lt + k * 32, 32, slabs[k % 2])
        write_slab(base_lt + k * 32, 32, slabs[k % 2])
        if k >= 1:
            drain_writes(base_lt + (k - 1) * 32, 32, slabs[(k - 1) % 2])
            nxt = k + 1
            if nxt < n_slabs:
                read_slab(base_lt + nxt * 32, 32, slabs[(k - 1) % 2])
            elif nxt == n_slabs and rem_lt:
                read_slab(base_lt + nxt * 32, rem_lt, slabs[(k - 1) % 2])
    drain_writes(base_lt + (n_slabs - 1) * 32, 32, slabs[(n_slabs - 1) % 2])
    if rem_lt:
        wait_read(base_lt + n_slabs * 32, rem_lt, slabs[n_slabs % 2])
        write_slab(base_lt + n_slabs * 32, rem_lt, slabs[n_slabs % 2])
        drain_writes(base_lt + n_slabs * 32, rem_lt, slabs[n_slabs % 2])

    @pl.when(wid == 0)
    def _extra():
        lt0 = _NW * _LT_PER_W
        pltpu.sync_copy(
            table_t_hbm.at[:, pl.ds(lt0 * 128, _EXTRA_LT * 128)],
            slab0_v.at[:, pl.ds(0, _EXTRA_LT * 128)],
        )
        write_slab(lt0, _EXTRA_LT, slab0_v)
        drain_writes(lt0, _EXTRA_LT, slab0_v)


@functools.partial(
    pl.kernel,
    mesh=_mesh,
    out_type=jax.ShapeDtypeStruct((_LATENT_DIM, _BATCH), jnp.float32),
    scratch_types=[
        pltpu.VMEM((_B_PER_W,), jnp.int32),
        pltpu.VMEM((_LATENT_DIM * _B_PER_W,), jnp.int32),
        pltpu.VMEM((_LATENT_DIM, _B_PER_W), jnp.float32),
        pltpu.VMEM(((_NUM_ACTIONS - _TAIL_START) * _LATENT_DIM,), jnp.float32),
        pltpu.SemaphoreType.DMA,
    ],
    compiler_params=pltpu.CompilerParams(
        use_tc_tiling_on_sc=False,
        needs_layout_passes=False,
        skip_device_barrier=True,
    ),
)
def _gather_kernel(idx_hbm, table_flat_hbm, tail_hbm, out_t_hbm, idx_v,
                   offs_v, block_v, tail_v, sem):
    wid = lax.axis_index("s") * _NC + lax.axis_index("c")
    base = wid * _B_PER_W
    pltpu.sync_copy(idx_hbm.at[pl.ds(base, _B_PER_W)], idx_v)
    pltpu.sync_copy(tail_hbm, tail_v)

    n_chunks = _B_PER_W // _L

    @pl.loop(0, n_chunks)
    def _compute_offsets(k):
        r = idx_v[pl.ds(k * _L, _L)]
        safe = jnp.where(r >= _TAIL_START, 0, r)
        base_off = ((safe >> 7) << 11) | (safe & 127)
        for c in range(_LATENT_DIM):
            offs_v[pl.ds(c * _B_PER_W + k * _L, _L)] = (
                base_off + ((c // 8) * 1024 + (c % 8) * 128)
            )

    for c in range(_LATENT_DIM):
        pltpu.async_copy(
            table_flat_hbm.at[offs_v.at[pl.ds(c * _B_PER_W, _B_PER_W)]],
            block_v.at[c],
            sem,
        )
    for c in range(_LATENT_DIM):
        pltpu.make_async_copy(
            table_flat_hbm.at[offs_v.at[pl.ds(c * _B_PER_W, _B_PER_W)]],
            block_v.at[c],
            sem,
        ).wait()

    @pl.loop(0, n_chunks)
    def _patch_tail(k):
        r = idx_v[pl.ds(k * _L, _L)]
        m = r >= _TAIL_START
        tr = jnp.where(m, r - _TAIL_START, 0)
        pos = k * _L + lax.iota(jnp.int32, _L)
        for c in range(_LATENT_DIM):
            vals = plsc.load_gather(tail_v, [tr * _LATENT_DIM + c], mask=m)
            plsc.store_scatter(
                block_v, [jnp.full((_L,), c, jnp.int32), pos], vals, mask=m
            )

    pltpu.sync_copy(block_v, out_t_hbm.at[:, pl.ds(base, _B_PER_W)])


def kernel(input, action_embedding):
    idx = input.reshape(_BATCH)
    flat = _detile_kernel(action_embedding.T).reshape(_FLAT_WORDS)
    tail = action_embedding[_TAIL_START:].reshape(-1)
    out_t = _gather_kernel(idx, flat, tail)
    return out_t.T[None, :, :]


# speedup vs baseline: 1.0611x; 1.0611x over previous
import functools
import jax
import jax.numpy as jnp
from jax import lax
from jax.experimental import pallas as pl
from jax.experimental.pallas import tpu as pltpu
from jax.experimental.pallas import tpu_sc as plsc

_NUM_ACTIONS = 1000000
_LATENT_DIM = 16
_BATCH = 16384

_NC = 2
_NS = 16
_NW = _NC * _NS
_B_PER_W = _BATCH // _NW
_L = 16

_FULL_LT = _NUM_ACTIONS // 128
_LT_PER_W = _FULL_LT // _NW
_EXTRA_LT = _FULL_LT - _NW * _LT_PER_W
_TAIL_START = _FULL_LT * 128
_FLAT_WORDS = _FULL_LT * _LATENT_DIM * 128

_mesh = plsc.VectorSubcoreMesh(core_axis_name="c", subcore_axis_name="s")


@functools.partial(
    pl.kernel,
    mesh=_mesh,
    out_type=jax.ShapeDtypeStruct((_FULL_LT, _LATENT_DIM, 128), jnp.float32),
    scratch_types=[
        pltpu.VMEM((_LATENT_DIM, 4096), jnp.float32),
        pltpu.VMEM((_LATENT_DIM, 4096), jnp.float32),
        pltpu.SemaphoreType.DMA,
        pltpu.SemaphoreType.DMA,
    ],
    compiler_params=pltpu.CompilerParams(
        needs_layout_passes=False, skip_device_barrier=True
    ),
)
def _detile_kernel(table_t_hbm, flat_hbm, slab0_v, slab1_v, rsem, wsem):
    wid = lax.axis_index("s") * _NC + lax.axis_index("c")
    base_lt = wid * _LT_PER_W
    n_slabs = _LT_PER_W // 32
    rem_lt = _LT_PER_W - 32 * n_slabs
    slabs = [slab0_v, slab1_v]

    def read_slab(lt0, n_lt, buf):
        pltpu.async_copy(
            table_t_hbm.at[:, pl.ds(lt0 * 128, n_lt * 128)],
            buf.at[:, pl.ds(0, n_lt * 128)],
            rsem,
        )

    def wait_read(lt0, n_lt, buf):
        pltpu.make_async_copy(
            table_t_hbm.at[:, pl.ds(lt0 * 128, n_lt * 128)],
            buf.at[:, pl.ds(0, n_lt * 128)],
            rsem,
        ).wait()

    def write_slab(lt0, n_lt, buf):
        for j in range(n_lt):
            pltpu.async_copy(
                buf.at[:, pl.ds(j * 128, 128)], flat_hbm.at[lt0 + j], wsem
            )

    def drain_writes(lt0, n_lt, buf):
        for j in range(n_lt):
            pltpu.make_async_copy(
                buf.at[:, pl.ds(j * 128, 128)], flat_hbm.at[lt0 + j], wsem
            ).wait()

    read_slab(base_lt, 32, slabs[0])
    for k in range(n_slabs):
        nxt = k + 1
        if nxt < n_slabs:
            read_slab(base_lt + nxt * 32, 32, slabs[nxt % 2])
        elif rem_lt:
            read_slab(base_lt + nxt * 32, rem_lt, slabs[nxt % 2])
        wait_read(base_lt + k * 32, 32, slabs[k % 2])
        write_slab(base_lt + k * 32, 32, slabs[k % 2])
        drain_writes(base_lt + k * 32, 32, slabs[k % 2])
    if rem_lt:
        wait_read(base_lt + n_slabs * 32, rem_lt, slabs[n_slabs % 2])
        write_slab(base_lt + n_slabs * 32, rem_lt, slabs[n_slabs % 2])
        drain_writes(base_lt + n_slabs * 32, rem_lt, slabs[n_slabs % 2])

    @pl.when(wid == 0)
    def _extra():
        lt0 = _NW * _LT_PER_W
        pltpu.sync_copy(
            table_t_hbm.at[:, pl.ds(lt0 * 128, _EXTRA_LT * 128)],
            slab0_v.at[:, pl.ds(0, _EXTRA_LT * 128)],
        )
        write_slab(lt0, _EXTRA_LT, slab0_v)
        drain_writes(lt0, _EXTRA_LT, slab0_v)


@functools.partial(
    pl.kernel,
    mesh=_mesh,
    out_type=jax.ShapeDtypeStruct((_LATENT_DIM, _BATCH), jnp.float32),
    scratch_types=[
        pltpu.VMEM((_B_PER_W,), jnp.int32),
        pltpu.VMEM((_LATENT_DIM * _B_PER_W,), jnp.int32),
        pltpu.VMEM((_LATENT_DIM, _B_PER_W), jnp.float32),
        pltpu.VMEM(((_NUM_ACTIONS - _TAIL_START) * _LATENT_DIM,), jnp.float32),
        pltpu.SemaphoreType.DMA,
    ],
    compiler_params=pltpu.CompilerParams(
        use_tc_tiling_on_sc=False,
        needs_layout_passes=False,
        skip_device_barrier=True,
    ),
)
def _gather_kernel(idx_hbm, table_flat_hbm, tail_hbm, out_t_hbm, idx_v,
                   offs_v, block_v, tail_v, sem):
    wid = lax.axis_index("s") * _NC + lax.axis_index("c")
    base = wid * _B_PER_W
    pltpu.sync_copy(idx_hbm.at[pl.ds(base, _B_PER_W)], idx_v)
    pltpu.sync_copy(tail_hbm, tail_v)

    n_chunks = _B_PER_W // _L

    @pl.loop(0, n_chunks)
    def _compute_offsets(k):
        r = idx_v[pl.ds(k * _L, _L)]
        safe = jnp.where(r >= _TAIL_START, 0, r)
        base_off = ((safe >> 7) << 11) | (safe & 127)
        for c in range(_LATENT_DIM):
            offs_v[pl.ds(c * _B_PER_W + k * _L, _L)] = (
                base_off + ((c // 8) * 1024 + (c % 8) * 128)
            )

    for c in range(_LATENT_DIM):
        pltpu.async_copy(
            table_flat_hbm.at[offs_v.at[pl.ds(c * _B_PER_W, _B_PER_W)]],
            block_v.at[c],
            sem,
        )
    for c in range(_LATENT_DIM):
        pltpu.make_async_copy(
            table_flat_hbm.at[offs_v.at[pl.ds(c * _B_PER_W, _B_PER_W)]],
            block_v.at[c],
            sem,
        ).wait()

    @pl.loop(0, n_chunks)
    def _patch_tail(k):
        r = idx_v[pl.ds(k * _L, _L)]
        m = r >= _TAIL_START
        tr = jnp.where(m, r - _TAIL_START, 0)
        pos = k * _L + lax.iota(jnp.int32, _L)
        for c in range(_LATENT_DIM):
            vals = plsc.load_gather(tail_v, [tr * _LATENT_DIM + c], mask=m)
            plsc.store_scatter(
                block_v, [jnp.full((_L,), c, jnp.int32), pos], vals, mask=m
            )

    pltpu.sync_copy(block_v, out_t_hbm.at[:, pl.ds(base, _B_PER_W)])


def kernel(input, action_embedding):
    idx = input.reshape(_BATCH)
    flat = _detile_kernel(action_embedding.T).reshape(_FLAT_WORDS)
    tail = action_embedding[_TAIL_START:].reshape(-1)
    out_t = _gather_kernel(idx, flat, tail)
    return out_t.T[None, :, :]
